# baseline (speedup 1.0000x reference)
import jax
import jax.numpy as jnp
from jax import lax
from jax.experimental import pallas as pl
from jax.experimental.pallas import tpu as pltpu
from jax.experimental.pallas import tpu_sc as plsc

NC = 2
NS = 16
L = 16
NW = NC * NS
EC = 128


def _make_sc_agg(n_pad, d, e_pad, first):
    ep_tile = e_pad // NW
    n_chunk = ep_tile // EC
    hc = n_chunk // 2
    hp = hc // 2
    rows_tile = n_pad // NS
    n_row_chunks = rows_tile // EC

    mesh = plsc.VectorSubcoreMesh(core_axis_name="c", subcore_axis_name="s")
    out_type = [jax.ShapeDtypeStruct((NC, n_pad, d), jnp.float32)]
    scratch = [
        pltpu.VMEM((hc, EC), jnp.int32),
        pltpu.VMEM((hc, EC), jnp.int32),
        pltpu.VMEM((EC, d), jnp.float32),
        pltpu.VMEM((EC, d), jnp.float32),
        pltpu.VMEM((EC,), jnp.float32),
        pltpu.VMEM((rows_tile,), jnp.float32),
        pltpu.VMEM_SHARED((n_pad, d), jnp.float32),
        pltpu.SemaphoreType.DMA,
        pltpu.SemaphoreType.DMA,
    ]
    if first:
        out_type.append(jax.ShapeDtypeStruct((NC, n_pad), jnp.float32))
        scratch += [
            pltpu.VMEM_SHARED((n_pad,), jnp.float32),
            pltpu.SemaphoreType.DMA,
        ]

    def body(h, src3, dst3, agg_out, deg_out,
             src2d, dst2d, rowsA, rowsB, onesbuf, degbuf, agg_sh,
             semG, semS, deg_sh=None, semD=None):
        c = lax.axis_index("c")
        s = lax.axis_index("s")
        wid = c * NS + s
        rbase = s * rows_tile
        zeros = jnp.zeros((L,), jnp.float32)

        def zrow(r, carry):
            for j in range(d // L):
                rowsA[r, pl.ds(j * L, L)] = zeros
            return carry
        lax.fori_loop(0, EC, zrow, 0)
        for j in range(EC // L):
            onesbuf[pl.ds(j * L, L)] = jnp.ones((L,), jnp.float32)
        for k in range(n_row_chunks):
            pltpu.sync_copy(rowsA, agg_sh.at[pl.ds(rbase + k * EC, EC), :])
        if first:
            def zr(i, carry):
                degbuf[pl.ds(i * L, L)] = zeros
                return carry
            lax.fori_loop(0, rows_tile // L, zr, 0)
            pltpu.sync_copy(degbuf, deg_sh.at[pl.ds(rbase, rows_tile)])
        plsc.subcore_barrier()

        def pair(t, carry):
            a = 2 * t
            b = a + 1
            pltpu.make_async_copy(h.at[src2d.at[a]], rowsA, semG).wait()
            gb = pltpu.async_copy(h.at[src2d.at[b]], rowsB, semG)
            sa = pltpu.async_copy(rowsA, agg_sh.at[dst2d.at[a]], semS,
                                  add=True)
            if first:
                @pl.when(t > 0)
                def _():
                    pltpu.make_async_copy(
                        onesbuf, deg_sh.at[dst2d.at[a - 2]], semD).wait()
                    pltpu.make_async_copy(
                        onesbuf, deg_sh.at[dst2d.at[b - 2]], semD).wait()
                pltpu.async_copy(onesbuf, deg_sh.at[dst2d.at[a]], semD,
                                 add=True)
                pltpu.async_copy(onesbuf, deg_sh.at[dst2d.at[b]], semD,
                                 add=True)
            gb.wait()
            sa.wait()

            @pl.when(t < hp - 1)
            def _():
                pltpu.async_copy(h.at[src2d.at[a + 2]], rowsA, semG)
            sb = pltpu.async_copy(rowsB, agg_sh.at[dst2d.at[b]], semS,
                                  add=True)
            sb.wait()
            return carry

        for half in range(2):
            base = half * hc
            pltpu.sync_copy(src3.at[wid, pl.ds(base, hc), :], src2d)
            pltpu.sync_copy(dst3.at[wid, pl.ds(base, hc), :], dst2d)
            pltpu.async_copy(h.at[src2d.at[0]], rowsA, semG)
            lax.fori_loop(0, hp, pair, 0)
            if first:
                last = hc - 2
                pltpu.make_async_copy(
                    onesbuf, deg_sh.at[dst2d.at[last]], semD).wait()
                pltpu.make_async_copy(
                    onesbuf, deg_sh.at[dst2d.at[last + 1]], semD).wait()
        plsc.subcore_barrier()

        pltpu.sync_copy(agg_sh.at[pl.ds(rbase, rows_tile), :],
                        agg_out.at[c, pl.ds(rbase, rows_tile), :])
        if first:
            pltpu.sync_copy(deg_sh.at[pl.ds(rbase, rows_tile)],
                            deg_out.at[c, pl.ds(rbase, rows_tile)])

    if first:
        def body_first(h, src3, dst3, agg_out, deg_out, *rest):
            return body(h, src3, dst3, agg_out, deg_out, *rest)
        fn = body_first
    else:
        def body_rest(h, src3, dst3, agg_out, *rest):
            return body(h, src3, dst3, agg_out, None, *rest)
        fn = body_rest

    return pl.kernel(fn, out_type=out_type, mesh=mesh, scratch_types=scratch)


def _make_tc_dense(n_pad, d, bsz):
    def tc_body(h_ref, m0_ref, m1_ref, d0_ref, d1_ref, ws_ref, wn_ref,
                b_ref, o_ref):
        recip = 1.0 / jnp.maximum(d0_ref[...] + d1_ref[...], 1.0)
        mean = (m0_ref[...] + m1_ref[...]) * recip
        acc = jnp.dot(h_ref[...], ws_ref[...],
                      preferred_element_type=jnp.float32)
        acc = acc + jnp.dot(mean, wn_ref[...],
                            preferred_element_type=jnp.float32)
        o_ref[...] = jnp.maximum(acc + b_ref[...], 0.0)

    return pl.pallas_call(
        tc_body,
        grid=(n_pad // bsz,),
        in_specs=[
            pl.BlockSpec((bsz, d), lambda i: (i, 0)),
            pl.BlockSpec((bsz, d), lambda i: (i, 0)),
            pl.BlockSpec((bsz, d), lambda i: (i, 0)),
            pl.BlockSpec((bsz, 1), lambda i: (i, 0)),
            pl.BlockSpec((bsz, 1), lambda i: (i, 0)),
            pl.BlockSpec((d, d), lambda i: (0, 0)),
            pl.BlockSpec((d, d), lambda i: (0, 0)),
            pl.BlockSpec((1, d), lambda i: (0, 0)),
        ],
        out_specs=pl.BlockSpec((bsz, d), lambda i: (i, 0)),
        out_shape=jax.ShapeDtypeStruct((n_pad, d), jnp.float32),
    )


def kernel(in_feat, edge_index, W_self1, W_neigh1, b1,
           W_self2, W_neigh2, b2, W_self3, W_neigh3, b3):
    n, d = in_feat.shape
    e = edge_index.shape[1]
    row_quant = NS * EC
    n_pad = ((n + row_quant - 1) // row_quant) * row_quant
    edge_quant = NW * EC * 4
    e_pad = ((e + edge_quant - 1) // edge_quant) * edge_quant
    n_chunk = e_pad // (NW * EC)

    src = edge_index[0].astype(jnp.int32)
    dst = edge_index[1].astype(jnp.int32)
    src = jnp.concatenate([src, jnp.zeros((e_pad - e,), jnp.int32)])
    dst = jnp.concatenate([dst, jnp.full((e_pad - e,), n, jnp.int32)])
    src3 = src.reshape(NW, n_chunk, EC)
    dst3 = dst.reshape(NW, n_chunk, EC)
    x = jnp.pad(in_feat, ((0, n_pad - n), (0, 0)))

    sc_first = _make_sc_agg(n_pad, d, e_pad, first=True)
    sc_rest = _make_sc_agg(n_pad, d, e_pad, first=False)
    tc = _make_tc_dense(n_pad, d, bsz=512)

    m, deg = sc_first(x, src3, dst3)
    d0 = deg[0].reshape(n_pad, 1)
    d1 = deg[1].reshape(n_pad, 1)
    h = tc(x, m[0], m[1], d0, d1, W_self1, W_neigh1, b1.reshape(1, d))
    [m] = sc_rest(h, src3, dst3)
    h = tc(h, m[0], m[1], d0, d1, W_self2, W_neigh2, b2.reshape(1, d))
    [m] = sc_rest(h, src3, dst3)
    h = tc(h, m[0], m[1], d0, d1, W_self3, W_neigh3, b3.reshape(1, d))
    return h[:n]


# speedup vs baseline: 1.0643x; 1.0643x over previous
import jax
import jax.numpy as jnp
from jax import lax
from jax.experimental import pallas as pl
from jax.experimental.pallas import tpu as pltpu
from jax.experimental.pallas import tpu_sc as plsc

NC = 2
NS = 16
L = 16
NW = NC * NS
EC = 128
SC0_SHARE = 0.8
STAGE_MAX = 32


def _pick_stages(k):
    for st in range(1, k + 1):
        if k % st == 0 and (k // st) % 2 == 0 and k // st <= STAGE_MAX:
            return st
    return k


def _make_sc_agg(n_pad, d, e_pad, first):
    t_chunks = e_pad // (NS * EC)
    k0 = int(t_chunks * SC0_SHARE) // 4 * 4
    k1 = t_chunks - k0
    st0 = _pick_stages(k0)
    st1 = _pick_stages(k1)
    slab = max(k0 // st0, k1 // st1)
    rows_tile = n_pad // NS
    n_row_chunks = rows_tile // EC

    mesh = plsc.VectorSubcoreMesh(core_axis_name="c", subcore_axis_name="s")
    out_type = [jax.ShapeDtypeStruct((NC, n_pad, d), jnp.float32)]
    scratch = [
        pltpu.VMEM((slab, EC), jnp.int32),
        pltpu.VMEM((slab, EC), jnp.int32),
        pltpu.VMEM((EC, d), jnp.float32),
        pltpu.VMEM((EC, d), jnp.float32),
        pltpu.VMEM((EC,), jnp.float32),
        pltpu.VMEM((rows_tile,), jnp.float32),
        pltpu.VMEM_SHARED((n_pad, d), jnp.float32),
        pltpu.SemaphoreType.DMA,
        pltpu.SemaphoreType.DMA,
    ]
    if first:
        out_type.append(jax.ShapeDtypeStruct((NC, n_pad), jnp.float32))
        scratch += [
            pltpu.VMEM_SHARED((n_pad,), jnp.float32),
            pltpu.SemaphoreType.DMA,
        ]

    def body(h, src2, dst2, agg_out, deg_out,
             src2d, dst2d, rowsA, rowsB, onesbuf, degbuf, agg_sh,
             semG, semS, deg_sh=None, semD=None):
        c = lax.axis_index("c")
        s = lax.axis_index("s")
        rbase = s * rows_tile
        zeros = jnp.zeros((L,), jnp.float32)

        def zrow(r, carry):
            for j in range(d // L):
                rowsA[r, pl.ds(j * L, L)] = zeros
            return carry
        lax.fori_loop(0, EC, zrow, 0)
        for j in range(EC // L):
            onesbuf[pl.ds(j * L, L)] = jnp.ones((L,), jnp.float32)
        for k in range(n_row_chunks):
            pltpu.sync_copy(rowsA, agg_sh.at[pl.ds(rbase + k * EC, EC), :])
        if first:
            def zr(i, carry):
                degbuf[pl.ds(i * L, L)] = zeros
                return carry
            lax.fori_loop(0, rows_tile // L, zr, 0)
            pltpu.sync_copy(degbuf, deg_sh.at[pl.ds(rbase, rows_tile)])
        plsc.subcore_barrier()

        def make_pair(sp):
            def pair(t, carry):
                a = 2 * t
                b = a + 1
                pltpu.make_async_copy(h.at[src2d.at[a]], rowsA, semG).wait()
                gb = pltpu.async_copy(h.at[src2d.at[b]], rowsB, semG)
                sa = pltpu.async_copy(rowsA, agg_sh.at[dst2d.at[a]], semS,
                                      add=True)
                if first:
                    @pl.when(t > 0)
                    def _():
                        pltpu.make_async_copy(
                            onesbuf, deg_sh.at[dst2d.at[a - 2]], semD).wait()
                        pltpu.make_async_copy(
                            onesbuf, deg_sh.at[dst2d.at[b - 2]], semD).wait()
                    pltpu.async_copy(onesbuf, deg_sh.at[dst2d.at[a]], semD,
                                     add=True)
                    pltpu.async_copy(onesbuf, deg_sh.at[dst2d.at[b]], semD,
                                     add=True)
                gb.wait()
                sa.wait()

                @pl.when(t < sp - 1)
                def _():
                    pltpu.async_copy(h.at[src2d.at[a + 2]], rowsA, semG)
                sb = pltpu.async_copy(rowsB, agg_sh.at[dst2d.at[b]], semS,
                                      add=True)
                sb.wait()
                return carry
            return pair

        def run_stream(chunk_base, k, n_stages):
            sc_ = k // n_stages
            sp = sc_ // 2
            pair = make_pair(sp)
            for stage in range(n_stages):
                base = chunk_base + stage * sc_
                pltpu.sync_copy(src2.at[pl.ds(base, sc_), :],
                                src2d.at[pl.ds(0, sc_), :])
                pltpu.sync_copy(dst2.at[pl.ds(base, sc_), :],
                                dst2d.at[pl.ds(0, sc_), :])
                pltpu.async_copy(h.at[src2d.at[0]], rowsA, semG)
                lax.fori_loop(0, sp, pair, 0)
                if first:
                    pltpu.make_async_copy(
                        onesbuf, deg_sh.at[dst2d.at[sc_ - 2]], semD).wait()
                    pltpu.make_async_copy(
                        onesbuf, deg_sh.at[dst2d.at[sc_ - 1]], semD).wait()

        @pl.when(c == 0)
        def _():
            run_stream(s * k0, k0, st0)

        @pl.when(c == 1)
        def _():
            run_stream(NS * k0 + s * k1, k1, st1)
        plsc.subcore_barrier()

        pltpu.sync_copy(agg_sh.at[pl.ds(rbase, rows_tile), :],
                        agg_out.at[c, pl.ds(rbase, rows_tile), :])
        if first:
            pltpu.sync_copy(deg_sh.at[pl.ds(rbase, rows_tile)],
                            deg_out.at[c, pl.ds(rbase, rows_tile)])

    if first:
        def body_first(h, src2, dst2, agg_out, deg_out, *rest):
            return body(h, src2, dst2, agg_out, deg_out, *rest)
        fn = body_first
    else:
        def body_rest(h, src2, dst2, agg_out, *rest):
            return body(h, src2, dst2, agg_out, None, *rest)
        fn = body_rest

    return pl.kernel(fn, out_type=out_type, mesh=mesh, scratch_types=scratch)


def _make_tc_dense(n_pad, d, bsz):
    def tc_body(h_ref, m0_ref, m1_ref, d0_ref, d1_ref, ws_ref, wn_ref,
                b_ref, o_ref):
        recip = 1.0 / jnp.maximum(d0_ref[...] + d1_ref[...], 1.0)
        mean = (m0_ref[...] + m1_ref[...]) * recip
        acc = jnp.dot(h_ref[...], ws_ref[...],
                      preferred_element_type=jnp.float32)
        acc = acc + jnp.dot(mean, wn_ref[...],
                            preferred_element_type=jnp.float32)
        o_ref[...] = jnp.maximum(acc + b_ref[...], 0.0)

    return pl.pallas_call(
        tc_body,
        grid=(n_pad // bsz,),
        in_specs=[
            pl.BlockSpec((bsz, d), lambda i: (i, 0)),
            pl.BlockSpec((bsz, d), lambda i: (i, 0)),
            pl.BlockSpec((bsz, d), lambda i: (i, 0)),
            pl.BlockSpec((bsz, 1), lambda i: (i, 0)),
            pl.BlockSpec((bsz, 1), lambda i: (i, 0)),
            pl.BlockSpec((d, d), lambda i: (0, 0)),
            pl.BlockSpec((d, d), lambda i: (0, 0)),
            pl.BlockSpec((1, d), lambda i: (0, 0)),
        ],
        out_specs=pl.BlockSpec((bsz, d), lambda i: (i, 0)),
        out_shape=jax.ShapeDtypeStruct((n_pad, d), jnp.float32),
    )


def kernel(in_feat, edge_index, W_self1, W_neigh1, b1,
           W_self2, W_neigh2, b2, W_self3, W_neigh3, b3):
    n, d = in_feat.shape
    e = edge_index.shape[1]
    row_quant = NS * EC
    n_pad = ((n + row_quant - 1) // row_quant) * row_quant
    edge_quant = NS * EC * 4
    e_pad = ((e + edge_quant - 1) // edge_quant) * edge_quant
    t_chunks = e_pad // (NS * EC)

    src = edge_index[0].astype(jnp.int32)
    dst = edge_index[1].astype(jnp.int32)
    src = jnp.concatenate([src, jnp.zeros((e_pad - e,), jnp.int32)])
    dst = jnp.concatenate([dst, jnp.full((e_pad - e,), n, jnp.int32)])
    src2 = src.reshape(NS * t_chunks, EC)
    dst2 = dst.reshape(NS * t_chunks, EC)
    x = jnp.pad(in_feat, ((0, n_pad - n), (0, 0)))

    sc_first = _make_sc_agg(n_pad, d, e_pad, first=True)
    sc_rest = _make_sc_agg(n_pad, d, e_pad, first=False)
    tc = _make_tc_dense(n_pad, d, bsz=512)

    m, deg = sc_first(x, src2, dst2)
    d0 = deg[0].reshape(n_pad, 1)
    d1 = deg[1].reshape(n_pad, 1)
    h = tc(x, m[0], m[1], d0, d1, W_self1, W_neigh1, b1.reshape(1, d))
    [m] = sc_rest(h, src2, dst2)
    h = tc(h, m[0], m[1], d0, d1, W_self2, W_neigh2, b2.reshape(1, d))
    [m] = sc_rest(h, src2, dst2)
    h = tc(h, m[0], m[1], d0, d1, W_self3, W_neigh3, b3.reshape(1, d))
    return h[:n]
